# baseline (speedup 1.0000x reference)
import functools

import jax
import jax.numpy as jnp
from jax import lax
from jax.experimental import pallas as pl
from jax.experimental.pallas import tpu as pltpu
from jax.experimental.pallas import tpu_sc as plsc

_BATCH = 4096
_HIST = 200
_DIM = 64
_N = _BATCH * _HIST
_V = 1000000

_NC = 2
_NS = 16
_NW = _NC * _NS

_TB = 128
_NBLK = _V // _TB
_TAIL = _V - _NBLK * _TB
_KT = _NBLK // _NW + 1

_CH = 128
_NCHUNK = _N // (_NW * _CH)
_NBUF = 4
_NGRP = _NCHUNK // _NBUF


def _transpose_body(embt_hbm, tail_hbm, wide_hbm, vin, vout, vtail, sem):
    wid = lax.axis_index("s") * _NC + lax.axis_index("c")
    lane = jax.lax.broadcasted_iota(jnp.int32, (16,), 0)
    rowidx = [16 * j + lane for j in range(4)]

    def transpose_block(src, nrows):
        def row(r, _):
            col = jnp.full((16,), 0, jnp.int32) + r
            for j in range(4):
                vals = plsc.load_gather(src, [rowidx[j], col])
                vout[r, pl.ds(16 * j, 16)] = vals
            return _

        lax.fori_loop(0, nrows, row, 0, unroll=16)

    def block(k, carry):
        g = k * _NW + wid

        @pl.when(g < _NBLK)
        def _do():
            c0 = g * _TB
            pltpu.async_copy(embt_hbm.at[:, pl.ds(c0, _TB)], vin, sem).wait()
            transpose_block(vin, _TB)
            pltpu.sync_copy(vout, wide_hbm.at[pl.ds(c0, _TB), :])

        return carry

    lax.fori_loop(0, _KT, block, 0)

    @pl.when(wid == _NW - 1)
    def _():
        pltpu.async_copy(tail_hbm.at[:, :], vtail, sem).wait()
        transpose_block(vtail, _TAIL)
        pltpu.sync_copy(
            vout.at[pl.ds(0, _TAIL), :], wide_hbm.at[pl.ds(_NBLK * _TB, _TAIL), :]
        )


def _gather_body(ids_hbm, wide_hbm, out_hbm, idx_v, rows, gsems, osems):
    wid = lax.axis_index("s") * _NC + lax.axis_index("c")
    pltpu.sync_copy(ids_hbm.at[wid], idx_v)

    def start_gather(slot, chunk):
        pltpu.async_copy(wide_hbm.at[idx_v.at[chunk]], rows[slot], gsems[slot])

    def wait_gather(slot, chunk):
        pltpu.make_async_copy(
            wide_hbm.at[idx_v.at[chunk]], rows[slot], gsems[slot]
        ).wait()

    def start_write(slot, chunk):
        base = (wid * _NCHUNK + chunk) * _CH
        pltpu.async_copy(
            rows[slot].at[:, pl.ds(0, _DIM)],
            out_hbm.at[pl.ds(base, _CH), pl.ds(0, _DIM)], osems[slot])

    def wait_write(slot, chunk):
        base = (wid * _NCHUNK + chunk) * _CH
        pltpu.make_async_copy(
            rows[slot].at[:, pl.ds(0, _DIM)],
            out_hbm.at[pl.ds(base, _CH), pl.ds(0, _DIM)], osems[slot]).wait()

    for b in range(_NBUF):
        start_gather(b, b)

    def group(g, _):
        for b in range(_NBUF):
            c = g * _NBUF + b
            wait_gather(b, c)
            start_write(b, c)
            wait_write(b, c)
            start_gather(b, c + _NBUF)
        return _

    lax.fori_loop(0, _NGRP - 1, group, 0)
    g_last = _NGRP - 1
    for b in range(_NBUF):
        c = g_last * _NBUF + b
        wait_gather(b, c)
        start_write(b, c)
        wait_write(b, c)


@jax.jit
def _run(ids3, embt, tail):
    mesh = plsc.VectorSubcoreMesh(core_axis_name="c", subcore_axis_name="s")
    transpose = functools.partial(
        pl.kernel,
        out_type=jax.ShapeDtypeStruct((_V, 128), jnp.float32),
        mesh=mesh,
        scratch_types=[
            pltpu.VMEM((_DIM, _TB), jnp.float32),
            pltpu.VMEM((_TB, 128), jnp.float32),
            pltpu.VMEM((_DIM, _TB), jnp.float32),
            pltpu.SemaphoreType.DMA,
        ],
        compiler_params=pltpu.CompilerParams(
            use_tc_tiling_on_sc=True, needs_layout_passes=False),
    )(_transpose_body)
    wide = transpose(embt, tail)
    gather = functools.partial(
        pl.kernel,
        out_type=jax.ShapeDtypeStruct((_N, 128), jnp.float32),
        mesh=mesh,
        scratch_types=[
            pltpu.VMEM((_NCHUNK, _CH), jnp.int32),
            [pltpu.VMEM((_CH, 128), jnp.float32) for _ in range(_NBUF)],
            [pltpu.SemaphoreType.DMA for _ in range(_NBUF)],
            [pltpu.SemaphoreType.DMA for _ in range(_NBUF)],
        ],
        compiler_params=pltpu.CompilerParams(use_tc_tiling_on_sc=False),
    )(_gather_body)
    return gather(ids3, wide)


def kernel(skill_ids, embeddings):
    ids3 = skill_ids.astype(jnp.int32).reshape(_NW, _NCHUNK, _CH)
    embt = embeddings.T
    tail_rows = embeddings[_NBLK * _TB:]
    tail = jnp.concatenate([tail_rows, tail_rows], axis=0).T
    out = _run(ids3, embt, tail)
    return out[:, :_DIM].reshape(_BATCH, _HIST, _DIM)


# speedup vs baseline: 1.2261x; 1.2261x over previous
import functools

import jax
import jax.numpy as jnp
from jax import lax
from jax.experimental import pallas as pl
from jax.experimental.pallas import tpu as pltpu
from jax.experimental.pallas import tpu_sc as plsc

_BATCH = 4096
_HIST = 200
_DIM = 64
_N = _BATCH * _HIST
_V = 1000000

_NC = 2
_NS = 16
_NW = _NC * _NS

_TB = 128
_NBLK = _V // _TB
_TAIL = _V - _NBLK * _TB
_KT = _NBLK // _NW + 1

_CH = 128
_NCHUNK = _N // (_NW * _CH)
_NBUF = 4
_NGRP = _NCHUNK // _NBUF


def _transpose_body(embt_hbm, tail_hbm, wide_hbm, vins, vouts, vtail, rsems,
                    wsems):
    wid = lax.axis_index("s") * _NC + lax.axis_index("c")
    lane = jax.lax.broadcasted_iota(jnp.int32, (16,), 0)
    rowidx = [16 * j + lane for j in range(4)]

    def transpose_block(src, dst, nrows):
        def row(r, _):
            col = jnp.full((16,), 0, jnp.int32) + r
            for j in range(4):
                vals = plsc.load_gather(src, [rowidx[j], col])
                dst[r, pl.ds(16 * j, 16)] = vals
            return _

        lax.fori_loop(0, nrows, row, 0, unroll=16)

    def rd(slot, g):
        pltpu.async_copy(
            embt_hbm.at[:, pl.ds(g * _TB, _TB)], vins[slot], rsems[slot])

    def rd_wait(slot, g):
        pltpu.make_async_copy(
            embt_hbm.at[:, pl.ds(g * _TB, _TB)], vins[slot], rsems[slot]).wait()

    def wr(slot, g):
        pltpu.async_copy(
            vouts[slot], wide_hbm.at[pl.ds(g * _TB, _TB), :], wsems[slot])

    def wr_wait(slot, g):
        pltpu.make_async_copy(
            vouts[slot], wide_hbm.at[pl.ds(g * _TB, _TB), :], wsems[slot]).wait()

    @pl.when(wid < _NBLK)
    def _prime():
        rd(0, wid)

    def block(k, carry):
        g = k * _NW + wid
        gn = g + _NW

        for slot in range(2):
            @pl.when((lax.rem(k, 2) == slot) & (g < _NBLK))
            def _do(slot=slot, g=g, gn=gn):
                rd_wait(slot, g)

                @pl.when(gn < _NBLK)
                def _pref():
                    rd(1 - slot, gn)

                @pl.when(k >= 2)
                def _drain():
                    wr_wait(slot, g - 2 * _NW)

                transpose_block(vins[slot], vouts[slot], _TB)
                wr(slot, g)

        return carry

    lax.fori_loop(0, _KT, block, 0)
    for slot in range(2):
        last_k = _KT - 1 - ((_KT - 1 + slot) % 2)
        g_last = last_k * _NW + wid

        @pl.when(g_last < _NBLK)
        def _final(slot=slot, g_last=g_last):
            wr_wait(slot, g_last)

    @pl.when(wid == _NW - 1)
    def _():
        pltpu.async_copy(tail_hbm.at[:, :], vtail, rsems[0]).wait()
        transpose_block(vtail, vouts[0], _TAIL)
        pltpu.sync_copy(
            vouts[0].at[pl.ds(0, _TAIL), :],
            wide_hbm.at[pl.ds(_NBLK * _TB, _TAIL), :]
        )


def _gather_body(ids_hbm, wide_hbm, out_hbm, idx_v, rows, gsems, osems):
    wid = lax.axis_index("s") * _NC + lax.axis_index("c")
    pltpu.sync_copy(ids_hbm.at[wid], idx_v)

    def start_gather(slot, chunk):
        pltpu.async_copy(wide_hbm.at[idx_v.at[chunk]], rows[slot], gsems[slot])

    def wait_gather(slot, chunk):
        pltpu.make_async_copy(
            wide_hbm.at[idx_v.at[chunk]], rows[slot], gsems[slot]
        ).wait()

    def start_write(slot, chunk):
        base = (wid * _NCHUNK + chunk) * _CH
        pltpu.async_copy(
            rows[slot].at[:, pl.ds(0, _DIM)],
            out_hbm.at[pl.ds(base, _CH), pl.ds(0, _DIM)], osems[slot])

    def wait_write(slot, chunk):
        base = (wid * _NCHUNK + chunk) * _CH
        pltpu.make_async_copy(
            rows[slot].at[:, pl.ds(0, _DIM)],
            out_hbm.at[pl.ds(base, _CH), pl.ds(0, _DIM)], osems[slot]).wait()

    for b in range(_NBUF):
        start_gather(b, b)

    def group(g, _):
        for b in range(_NBUF):
            c = g * _NBUF + b
            wait_gather(b, c)
            start_write(b, c)
            wait_write(b, c)
            start_gather(b, c + _NBUF)
        return _

    lax.fori_loop(0, _NGRP - 1, group, 0)
    g_last = _NGRP - 1
    for b in range(_NBUF):
        c = g_last * _NBUF + b
        wait_gather(b, c)
        start_write(b, c)
        wait_write(b, c)


@jax.jit
def _run(ids3, embt, tail):
    mesh = plsc.VectorSubcoreMesh(core_axis_name="c", subcore_axis_name="s")
    transpose = functools.partial(
        pl.kernel,
        out_type=jax.ShapeDtypeStruct((_V, 128), jnp.float32),
        mesh=mesh,
        scratch_types=[
            [pltpu.VMEM((_DIM, _TB), jnp.float32) for _ in range(2)],
            [pltpu.VMEM((_TB, 128), jnp.float32) for _ in range(2)],
            pltpu.VMEM((_DIM, _TB), jnp.float32),
            [pltpu.SemaphoreType.DMA for _ in range(2)],
            [pltpu.SemaphoreType.DMA for _ in range(2)],
        ],
        compiler_params=pltpu.CompilerParams(
            use_tc_tiling_on_sc=True, needs_layout_passes=False),
    )(_transpose_body)
    wide = transpose(embt, tail)
    gather = functools.partial(
        pl.kernel,
        out_type=jax.ShapeDtypeStruct((_N, 128), jnp.float32),
        mesh=mesh,
        scratch_types=[
            pltpu.VMEM((_NCHUNK, _CH), jnp.int32),
            [pltpu.VMEM((_CH, 128), jnp.float32) for _ in range(_NBUF)],
            [pltpu.SemaphoreType.DMA for _ in range(_NBUF)],
            [pltpu.SemaphoreType.DMA for _ in range(_NBUF)],
        ],
        compiler_params=pltpu.CompilerParams(use_tc_tiling_on_sc=False),
    )(_gather_body)
    return gather(ids3, wide)


def kernel(skill_ids, embeddings):
    ids3 = skill_ids.astype(jnp.int32).reshape(_NW, _NCHUNK, _CH)
    embt = embeddings.T
    tail_rows = embeddings[_NBLK * _TB:]
    tail = jnp.concatenate([tail_rows, tail_rows], axis=0).T
    out = _run(ids3, embt, tail)
    return out[:, :_DIM].reshape(_BATCH, _HIST, _DIM)


# speedup vs baseline: 2.5136x; 2.0500x over previous
import functools

import jax
import jax.numpy as jnp
from jax import lax
from jax.experimental import pallas as pl
from jax.experimental.pallas import tpu as pltpu
from jax.experimental.pallas import tpu_sc as plsc

_BATCH = 4096
_HIST = 200
_DIM = 64
_N = _BATCH * _HIST

_NC = 2
_NS = 16
_NW = _NC * _NS
_CH = 128
_NCHUNK = _N // (_NW * _CH)
_NBUF = 8
_NGRP = _NCHUNK // _NBUF


def _gather_body(ids_hbm, table_hbm, out_hbm, idx_v, rows, gsems, osems):
    wid = lax.axis_index("s") * _NC + lax.axis_index("c")

    pltpu.sync_copy(ids_hbm.at[wid], idx_v)

    def start_gather(slot, chunk):
        pltpu.async_copy(table_hbm.at[idx_v.at[chunk]], rows[slot], gsems[slot])

    def wait_gather(slot, chunk):
        pltpu.make_async_copy(
            table_hbm.at[idx_v.at[chunk]], rows[slot], gsems[slot]
        ).wait()

    def start_write(slot, chunk):
        base = (wid * _NCHUNK + chunk) * _CH
        pltpu.async_copy(
            rows[slot], out_hbm.at[pl.ds(base, _CH), pl.ds(0, _DIM)],
            osems[slot])

    def wait_write(slot, chunk):
        base = (wid * _NCHUNK + chunk) * _CH
        pltpu.make_async_copy(
            rows[slot], out_hbm.at[pl.ds(base, _CH), pl.ds(0, _DIM)],
            osems[slot]).wait()

    for b in range(_NBUF):
        start_gather(b, b)

    def group(g, _):
        for b in range(_NBUF):
            c = g * _NBUF + b
            wait_gather(b, c)
            start_write(b, c)
            wait_write(b, c)
            start_gather(b, c + _NBUF)
        return _

    lax.fori_loop(0, _NGRP - 1, group, 0, unroll=False)

    g_last = _NGRP - 1
    for b in range(_NBUF):
        c = g_last * _NBUF + b
        wait_gather(b, c)
        start_write(b, c)
        wait_write(b, c)


@functools.partial(jax.jit, donate_argnums=())
def _run(ids3, table):
    mesh = plsc.VectorSubcoreMesh(core_axis_name="c", subcore_axis_name="s")
    f = functools.partial(
        pl.kernel,
        out_type=jax.ShapeDtypeStruct((_N, 2 * _DIM), jnp.float32),
        mesh=mesh,
        scratch_types=[
            pltpu.VMEM((_NCHUNK, _CH), jnp.int32),
            [pltpu.VMEM((_CH, _DIM), jnp.float32) for _ in range(_NBUF)],
            [pltpu.SemaphoreType.DMA for _ in range(_NBUF)],
            [pltpu.SemaphoreType.DMA for _ in range(_NBUF)],
        ],
        compiler_params=pltpu.CompilerParams(use_tc_tiling_on_sc=False),
    )(_gather_body)
    return f(ids3, table)


def kernel(skill_ids, embeddings):
    ids3 = skill_ids.astype(jnp.int32).reshape(_NW, _NCHUNK, _CH)
    out = _run(ids3, embeddings)
    return out[:, :_DIM].reshape(_BATCH, _HIST, _DIM)
